# baseline (speedup 1.0000x reference)
import functools

import jax
import jax.numpy as jnp
from jax import lax
from jax.experimental import pallas as pl
from jax.experimental.pallas import tpu as pltpu
from jax.experimental.pallas import tpu_sc as plsc

BATCH = 16384
DIM = 64
FIELDS = 4
LANES = 16

_NC = 2
_NS = 16
_NW = _NC * _NS
_B_PER_W = BATCH // _NW
_R = 256
_C = _B_PER_W // _R
_CG = DIM // LANES

_mesh = plsc.VectorSubcoreMesh(core_axis_name="c", subcore_axis_name="s")


@functools.partial(
    pl.kernel,
    out_type=jax.ShapeDtypeStruct((BATCH, DIM), jnp.float32),
    mesh=_mesh,
    scratch_types=[
        [pltpu.VMEM((_R,), jnp.int32) for _ in range(FIELDS)],
        [pltpu.VMEM((_R, DIM), jnp.float32) for _ in range(FIELDS)],
        pltpu.SemaphoreType.DMA,
    ],
    compiler_params=pltpu.CompilerParams(use_tc_tiling_on_sc=False),
)
def _embed_sum(l0, l1, l2, l3, t0, t1, t2, t3, out, idx_v, rows_v, sem):
    wid = lax.axis_index("s") * _NC + lax.axis_index("c")
    base = wid * _B_PER_W
    labels = [l0, l1, l2, l3]
    tables = [t0, t1, t2, t3]

    for c in range(_C):
        row0 = base + c * _R
        for f in range(FIELDS):
            pltpu.sync_copy(labels[f].at[pl.ds(row0, _R)], idx_v[f])
        descs = [
            pltpu.async_copy(tables[f].at[idx_v[f]], rows_v[f], sem)
            for f in range(FIELDS)
        ]
        for d in descs:
            d.wait()

        def body(r, carry):
            for cg in range(_CG):
                sl = pl.ds(cg * LANES, LANES)
                acc = (rows_v[0][r, sl] + rows_v[1][r, sl]
                       + rows_v[2][r, sl] + rows_v[3][r, sl])
                rows_v[0][r, sl] = acc
            return carry

        lax.fori_loop(0, _R, body, 0)
        pltpu.sync_copy(rows_v[0], out.at[pl.ds(row0, _R)])


def kernel(labels_f0, labels_f1, labels_f2, labels_f3,
           table_f0, table_f1, table_f2, table_f3):
    return _embed_sum(labels_f0, labels_f1, labels_f2, labels_f3,
                      table_f0, table_f1, table_f2, table_f3)


# speedup vs baseline: 1.0081x; 1.0081x over previous
import functools

import jax
import jax.numpy as jnp
from jax import lax
from jax.experimental import pallas as pl
from jax.experimental.pallas import tpu as pltpu
from jax.experimental.pallas import tpu_sc as plsc

BATCH = 16384
VOCABP1 = 100001
DIM = 64
FIELDS = 4
LANES = 16

_NC = 2
_NS = 16
_NW = _NC * _NS
_R = BATCH // _NW
_CG = DIM // LANES

_mesh = plsc.VectorSubcoreMesh(core_axis_name="c", subcore_axis_name="s")
_params = pltpu.CompilerParams(use_tc_tiling_on_sc=False)


@functools.partial(
    pl.kernel,
    out_type=jax.ShapeDtypeStruct((BATCH, DIM), jnp.float32),
    mesh=_mesh,
    scratch_types=[
        pltpu.VMEM((_R,), jnp.int32),
        pltpu.VMEM((_R, DIM), jnp.float32),
        pltpu.SemaphoreType.DMA,
    ],
    compiler_params=_params,
)
def _gather_first(lab, tab, out, idx_v, gbuf, sem):
    wid = lax.axis_index("s") * _NC + lax.axis_index("c")
    base = wid * _R
    pltpu.sync_copy(lab.at[pl.ds(base, _R)], idx_v)
    pltpu.async_copy(tab.at[idx_v], gbuf, sem).wait()
    pltpu.sync_copy(gbuf, out.at[pl.ds(base, _R)])


@functools.partial(
    pl.kernel,
    out_type=jax.ShapeDtypeStruct((BATCH, DIM), jnp.float32),
    mesh=_mesh,
    scratch_types=[
        pltpu.VMEM((_R,), jnp.int32),
        pltpu.VMEM((_R, DIM), jnp.float32),
        pltpu.VMEM((_R, DIM), jnp.float32),
        pltpu.SemaphoreType.DMA,
        pltpu.SemaphoreType.DMA,
    ],
    compiler_params=_params,
)
def _gather_acc(lab, tab, acc, out, idx_v, gbuf, abuf, gsem, asem):
    wid = lax.axis_index("s") * _NC + lax.axis_index("c")
    base = wid * _R
    pltpu.sync_copy(lab.at[pl.ds(base, _R)], idx_v)
    gd = pltpu.async_copy(tab.at[idx_v], gbuf, gsem)
    ad = pltpu.async_copy(acc.at[pl.ds(base, _R)], abuf, asem)
    gd.wait()
    ad.wait()

    def body(r, carry):
        for cg in range(_CG):
            sl = pl.ds(cg * LANES, LANES)
            gbuf[r, sl] = gbuf[r, sl] + abuf[r, sl]
        return carry

    lax.fori_loop(0, _R, body, 0)
    pltpu.sync_copy(gbuf, out.at[pl.ds(base, _R)])


def kernel(labels_f0, labels_f1, labels_f2, labels_f3,
           table_f0, table_f1, table_f2, table_f3):
    acc = _gather_first(labels_f0, table_f0)
    acc = _gather_acc(labels_f1, table_f1, acc)
    acc = _gather_acc(labels_f2, table_f2, acc)
    acc = _gather_acc(labels_f3, table_f3, acc)
    return acc
